# baseline (speedup 1.0000x reference)
import dataclasses

import jax
import jax.numpy as jnp
from jax.experimental import pallas as pl
from jax.experimental.pallas import tpu as pltpu
from jax.experimental.pallas import tpu_sc as plsc

K = 8
SC_BT = 16


def _to_monotone(bits):
    return bits ^ (jax.lax.shift_right_arithmetic(bits, 31) & 0x7FFFFFFF)


def _tc_logits_kernel(x_ref, wt_ref, b_ref, logits_ref, conf_ref,
                      *, n_tokens: int):
    i = pl.program_id(0)
    logits = jnp.dot(x_ref[...], wt_ref[...],
                     preferred_element_type=jnp.float32)
    logits = logits + b_ref[...]
    logits_ref[...] = logits

    @pl.when(i == 0)
    def _():
        conf_ref[0, 0] = 0.0

    rowmax = jnp.max(logits, axis=1, keepdims=True)
    conf_ref[0, 0] += jnp.sum(rowmax) * (1.0 / n_tokens)


def _sc_topk_token(l_vmem, t, num_experts):
    it = jax.lax.iota(jnp.int32, 16)
    kregs = []
    for c in range(num_experts // 16):
        r = l_vmem.at[t, pl.ds(16 * c, 16)][...]
        bits = jax.lax.bitcast_convert_type(r, jnp.int32)
        s = _to_monotone(bits)
        ks = (s & -64) | ((num_experts - 1 - 16 * c) - it)
        kf = jax.lax.bitcast_convert_type(_to_monotone(ks), jnp.float32)
        kregs.append(kf)

    lo8 = it < 8

    def sort16(v):
        sk, _ = plsc.sort_key_val(v, v, descending=True)
        return sk

    def top8_merge(a, b):
        m = jnp.where(lo8, a, jax.lax.rev(b, (0,)))
        return sort16(m)

    s0, s1, s2, s3 = [sort16(kf) for kf in kregs]
    s01 = top8_merge(s0, s1)
    s23 = top8_merge(s2, s3)
    final = top8_merge(s01, s23)
    return kregs, final


def _sc_gates_token(g_vmem, t, kregs, final, num_experts):
    it = jax.lax.iota(jnp.int32, 16)
    kmax = jnp.max(final)
    kth = jnp.min(jnp.where(it < K, final, jnp.float32(jnp.inf)))
    gs = []
    for kf in kregs:
        e = jnp.exp(kf - kmax)
        gs.append(jnp.where(kf >= kth, e, jnp.float32(0.0)))
    tot = gs[0]
    for g in gs[1:]:
        tot = tot + g
    denom = jnp.sum(tot)
    ones = jnp.full((16,), 1.0, jnp.float32)
    inv = ones / (ones * denom)
    for c, g in enumerate(gs):
        g_vmem.at[t, pl.ds(16 * c, 16)][...] = g * inv


def _sc_decode_idx(final, num_experts):
    sb = _to_monotone(jax.lax.bitcast_convert_type(final, jnp.int32))
    return (num_experts - 1) - (sb & (num_experts - 1))


def _sc_router(logits, num_experts):
    n_tokens = logits.shape[0]
    mesh = plsc.VectorSubcoreMesh(core_axis_name="core",
                                  subcore_axis_name="subcore")
    cp = pltpu.CompilerParams()
    if "needs_layout_passes" in pltpu.CompilerParams.__dataclass_fields__:
        cp = dataclasses.replace(cp, needs_layout_passes=False)

    @pl.kernel(
        out_type=[
            jax.ShapeDtypeStruct((n_tokens, num_experts), jnp.float32),
            jax.ShapeDtypeStruct((n_tokens, 16), jnp.int32),
        ],
        mesh=mesh,
        compiler_params=cp,
    )
    def k(logits_hbm, gates_hbm, idx_hbm):
        def body(l_vmem, g_vmem, i_vmem):
            @pl.loop(0, SC_BT)
            def _(t):
                kregs0, final0 = _sc_topk_token(l_vmem, t, num_experts)
                _sc_gates_token(g_vmem, t, kregs0, final0, num_experts)
                i_vmem.at[t, :][...] = _sc_decode_idx(final0, num_experts)

        pltpu.emit_pipeline(
            body,
            grid=(n_tokens // SC_BT,),
            in_specs=[pl.BlockSpec((SC_BT, num_experts),
                                   index_map=lambda i: (i, 0))],
            out_specs=[pl.BlockSpec((SC_BT, num_experts),
                                    index_map=lambda i: (i, 0)),
                       pl.BlockSpec((SC_BT, 16),
                                    index_map=lambda i: (i, 0))],
            core_axis_name=("core", "subcore"),
            dimension_semantics=(pltpu.PARALLEL,),
        )(logits_hbm, gates_hbm, idx_hbm)

    return k(logits)


def kernel(x, W, b):
    n_tokens, input_dim = x.shape
    num_experts = W.shape[0]
    bt = min(1024, n_tokens)
    grid = (n_tokens // bt,)

    wt = W.T
    b2 = b.reshape(1, num_experts)

    logits, conf = pl.pallas_call(
        lambda *refs: _tc_logits_kernel(*refs, n_tokens=n_tokens),
        grid=grid,
        in_specs=[
            pl.BlockSpec((bt, input_dim), lambda i: (i, 0)),
            pl.BlockSpec((input_dim, num_experts), lambda i: (0, 0)),
            pl.BlockSpec((1, num_experts), lambda i: (0, 0)),
        ],
        out_specs=[
            pl.BlockSpec((bt, num_experts), lambda i: (i, 0)),
            pl.BlockSpec((1, 1), lambda i: (0, 0),
                         memory_space=pltpu.SMEM),
        ],
        out_shape=[
            jax.ShapeDtypeStruct((n_tokens, num_experts), jnp.float32),
            jax.ShapeDtypeStruct((1, 1), jnp.float32),
        ],
        compiler_params=pltpu.CompilerParams(
            dimension_semantics=("arbitrary",)),
    )(x, wt, b2)

    gates, idx16 = _sc_router(logits, num_experts)
    return gates, jax.lax.slice(idx16, (0, 0), (n_tokens, K)), conf.reshape(())


# speedup vs baseline: 1.0756x; 1.0756x over previous
import dataclasses

import jax
import jax.numpy as jnp
from jax.experimental import pallas as pl
from jax.experimental.pallas import tpu as pltpu
from jax.experimental.pallas import tpu_sc as plsc

K = 8
SC_BT = 16


def _to_monotone(bits):
    return bits ^ (jax.lax.shift_right_arithmetic(bits, 31) & 0x7FFFFFFF)


def _tc_logits_kernel(x_ref, wt_ref, b_ref, logits_ref, conf_ref,
                      *, n_tokens: int):
    i = pl.program_id(0)
    logits = jnp.dot(x_ref[...], wt_ref[...],
                     preferred_element_type=jnp.float32)
    logits = logits + b_ref[...]
    logits_ref[...] = logits

    @pl.when(i == 0)
    def _():
        conf_ref[0, 0] = 0.0

    rowmax = jnp.max(logits, axis=1, keepdims=True)
    conf_ref[0, 0] += jnp.sum(rowmax) * (1.0 / n_tokens)


def _sc_topk_token(l_vmem, t, num_experts):
    it = jax.lax.iota(jnp.int32, 16)
    kregs = []
    for c in range(num_experts // 16):
        r = l_vmem.at[t, pl.ds(16 * c, 16)][...]
        bits = jax.lax.bitcast_convert_type(r, jnp.int32)
        s = _to_monotone(bits)
        ks = (s & -64) | ((num_experts - 1 - 16 * c) - it)
        kf = jax.lax.bitcast_convert_type(_to_monotone(ks), jnp.float32)
        kregs.append(kf)

    lo8 = it < 8

    def sort16(v):
        sk, _ = plsc.sort_key_val(v, v, descending=True)
        return sk

    def top8_merge(a, b):
        m = jnp.where(lo8, a, jax.lax.rev(b, (0,)))
        return sort16(m)

    s0, s1, s2, s3 = [sort16(kf) for kf in kregs]
    s01 = top8_merge(s0, s1)
    s23 = top8_merge(s2, s3)
    final = top8_merge(s01, s23)
    return kregs, final


def _sc_gates_token(g_vmem, t, kregs, final, num_experts):
    it = jax.lax.iota(jnp.int32, 16)
    kmax = jnp.max(final)
    kth = jnp.min(jnp.where(it < K, final, jnp.float32(jnp.inf)))
    gs = []
    for kf in kregs:
        e = jnp.exp(kf - kmax)
        gs.append(jnp.where(kf >= kth, e, jnp.float32(0.0)))
    tot = gs[0]
    for g in gs[1:]:
        tot = tot + g
    denom = jnp.sum(tot)
    ones = jnp.full((16,), 1.0, jnp.float32)
    inv = ones / (ones * denom)
    for c, g in enumerate(gs):
        g_vmem.at[t, pl.ds(16 * c, 16)][...] = g * inv


def _sc_decode_idx(final, num_experts):
    sb = _to_monotone(jax.lax.bitcast_convert_type(final, jnp.int32))
    return (num_experts - 1) - (sb & (num_experts - 1))


def _sc_router(logits, num_experts):
    n_tokens = logits.shape[0]
    mesh = plsc.VectorSubcoreMesh(core_axis_name="core",
                                  subcore_axis_name="subcore")
    cp = pltpu.CompilerParams()
    if "needs_layout_passes" in pltpu.CompilerParams.__dataclass_fields__:
        cp = dataclasses.replace(cp, needs_layout_passes=False)

    @pl.kernel(
        out_type=[
            jax.ShapeDtypeStruct((n_tokens, num_experts), jnp.float32),
            jax.ShapeDtypeStruct((n_tokens, 16), jnp.int32),
        ],
        mesh=mesh,
        compiler_params=cp,
    )
    def k(logits_hbm, gates_hbm, idx_hbm):
        def body(l_vmem, g_vmem, i_vmem):
            @pl.loop(0, SC_BT)
            def _(t):
                kregs0, final0 = _sc_topk_token(l_vmem, t, num_experts)
                _sc_gates_token(g_vmem, t, kregs0, final0, num_experts)
                i_vmem.at[t, :][...] = _sc_decode_idx(final0, num_experts)

        pltpu.emit_pipeline(
            body,
            grid=(n_tokens // SC_BT,),
            in_specs=[pl.BlockSpec((SC_BT, num_experts),
                                   index_map=lambda i: (i, 0))],
            out_specs=[pl.BlockSpec((SC_BT, num_experts),
                                    index_map=lambda i: (i, 0)),
                       pl.BlockSpec((SC_BT, 16),
                                    index_map=lambda i: (i, 0))],
            core_axis_name=("core", "subcore"),
            dimension_semantics=(pltpu.PARALLEL,),
        )(logits_hbm, gates_hbm, idx_hbm)

    return k(logits)


def kernel(x, W, b):
    n_tokens, input_dim = x.shape
    num_experts = W.shape[0]
    bt = min(1024, n_tokens)

    wt = W.T
    b2 = b.reshape(1, num_experts)

    n_chunks = 4
    nc = n_tokens // n_chunks
    blocks_per_chunk = nc // bt

    gates_parts = []
    idx_parts = []
    conf_parts = []
    for j in range(n_chunks):
        base = j * blocks_per_chunk
        logits, conf = pl.pallas_call(
            lambda *refs: _tc_logits_kernel(*refs, n_tokens=n_tokens),
            grid=(blocks_per_chunk,),
            in_specs=[
                pl.BlockSpec((bt, input_dim),
                             lambda i, base=base: (base + i, 0)),
                pl.BlockSpec((input_dim, num_experts), lambda i: (0, 0)),
                pl.BlockSpec((1, num_experts), lambda i: (0, 0)),
            ],
            out_specs=[
                pl.BlockSpec((bt, num_experts), lambda i: (i, 0)),
                pl.BlockSpec((1, 1), lambda i: (0, 0),
                             memory_space=pltpu.SMEM),
            ],
            out_shape=[
                jax.ShapeDtypeStruct((nc, num_experts), jnp.float32),
                jax.ShapeDtypeStruct((1, 1), jnp.float32),
            ],
            compiler_params=pltpu.CompilerParams(
                dimension_semantics=("arbitrary",)),
        )(x, wt, b2)
        g, i16 = _sc_router(logits, num_experts)
        gates_parts.append(g)
        idx_parts.append(jax.lax.slice(i16, (0, 0), (nc, K)))
        conf_parts.append(conf.reshape(()))

    gates = jnp.concatenate(gates_parts, axis=0)
    idx = jnp.concatenate(idx_parts, axis=0)
    conf = conf_parts[0]
    for c in conf_parts[1:]:
        conf = conf + c
    return gates, idx, conf


# speedup vs baseline: 1.3552x; 1.2600x over previous
import jax
import jax.numpy as jnp
from jax.experimental import pallas as pl
from jax.experimental.pallas import tpu as pltpu

K = 8


def _to_monotone(bits):
    return bits ^ (jax.lax.shift_right_arithmetic(bits, 31) & 0x7FFFFFFF)


def _router_kernel(x_ref, wt_ref, b_ref, gates_ref, idx_ref, conf_ref,
                   *, n_tokens: int, num_experts: int):
    i = pl.program_id(0)
    logits = jnp.dot(x_ref[...], wt_ref[...],
                     preferred_element_type=jnp.float32)
    logits = logits + b_ref[...]

    lane = jax.lax.broadcasted_iota(jnp.int32, logits.shape, 1)
    bits = jax.lax.bitcast_convert_type(logits, jnp.int32)
    s = _to_monotone(bits)
    key_s = (s & -64) | (num_experts - 1 - lane)
    keys = jax.lax.bitcast_convert_type(_to_monotone(key_s), jnp.float32)

    neg = jnp.float32(-jnp.inf)
    work = logits
    vals = []
    kbests = []
    for _ in range(K):
        m = jnp.max(work, axis=1, keepdims=True)
        kb = jnp.max(jnp.where(work == m, keys, neg), axis=1, keepdims=True)
        work = jnp.where(keys == kb, neg, work)
        vals.append(m)
        kbests.append(kb)

    m0 = vals[0]
    m8 = vals[K - 1]
    kb8 = kbests[K - 1]

    e = jnp.exp(logits - m0)
    sel = (logits > m8) | ((logits == m8) & (keys >= kb8))
    esel = jnp.where(sel, e, 0.0)
    denom = jnp.sum(esel, axis=1, keepdims=True)
    gates_ref[...] = esel * (1.0 / denom)

    kcat = jnp.concatenate(kbests, axis=1)
    sb = _to_monotone(jax.lax.bitcast_convert_type(kcat, jnp.int32))
    idx_ref[...] = (num_experts - 1) - (sb & (num_experts - 1))

    @pl.when(i == 0)
    def _():
        conf_ref[0, 0] = 0.0

    conf_ref[0, 0] += jnp.sum(m0) * (1.0 / n_tokens)


def kernel(x, W, b):
    n_tokens, input_dim = x.shape
    num_experts = W.shape[0]
    bt = min(1024, n_tokens)
    grid = (n_tokens // bt,)

    wt = W.T
    b2 = b.reshape(1, num_experts)

    gates, idx, conf = pl.pallas_call(
        lambda *refs: _router_kernel(*refs, n_tokens=n_tokens,
                                     num_experts=num_experts),
        grid=grid,
        in_specs=[
            pl.BlockSpec((bt, input_dim), lambda i: (i, 0)),
            pl.BlockSpec((input_dim, num_experts), lambda i: (0, 0)),
            pl.BlockSpec((1, num_experts), lambda i: (0, 0)),
        ],
        out_specs=[
            pl.BlockSpec((bt, num_experts), lambda i: (i, 0)),
            pl.BlockSpec((bt, K), lambda i: (i, 0)),
            pl.BlockSpec((1, 1), lambda i: (0, 0),
                         memory_space=pltpu.SMEM),
        ],
        out_shape=[
            jax.ShapeDtypeStruct((n_tokens, num_experts), jnp.float32),
            jax.ShapeDtypeStruct((n_tokens, K), jnp.int32),
            jax.ShapeDtypeStruct((1, 1), jnp.float32),
        ],
        compiler_params=pltpu.CompilerParams(
            dimension_semantics=("arbitrary",)),
    )(x, wt, b2)

    return gates, idx, conf.reshape(())


# speedup vs baseline: 1.3575x; 1.0017x over previous
import jax
import jax.numpy as jnp
from jax.experimental import pallas as pl
from jax.experimental.pallas import tpu as pltpu

K = 8


def _to_monotone(bits):
    return bits ^ (jax.lax.shift_right_arithmetic(bits, 31) & 0x7FFFFFFF)


def _router_kernel(x_ref, wt_ref, b_ref, gates_ref, idx_ref, conf_ref,
                   *, n_tokens: int, num_experts: int):
    i = pl.program_id(0)
    logits = jnp.dot(x_ref[...], wt_ref[...],
                     preferred_element_type=jnp.float32)
    logits = logits + b_ref[...]

    lane = jax.lax.broadcasted_iota(jnp.int32, logits.shape, 1)
    bits = jax.lax.bitcast_convert_type(logits, jnp.int32)
    s = _to_monotone(bits)
    key_s = (s & -64) | (num_experts - 1 - lane)
    keys = jax.lax.bitcast_convert_type(_to_monotone(key_s), jnp.float32)

    neg = jnp.float32(-jnp.inf)
    work = logits
    vals = []
    kbests = []
    for _ in range(K):
        m = jnp.max(work, axis=1, keepdims=True)
        kb = jnp.max(jnp.where(work == m, keys, neg), axis=1, keepdims=True)
        work = jnp.where(work == m, neg, work)
        vals.append(m)
        kbests.append(kb)

    m0 = vals[0]
    m8 = vals[K - 1]
    kb8 = kbests[K - 1]

    e = jnp.exp(logits - m0)
    sel = (logits > m8) | ((logits == m8) & (keys >= kb8))
    esel = jnp.where(sel, e, 0.0)
    denom = jnp.sum(esel, axis=1, keepdims=True)
    gates_ref[...] = esel * (1.0 / denom)

    kcat = jnp.concatenate(kbests, axis=1)
    sb = _to_monotone(jax.lax.bitcast_convert_type(kcat, jnp.int32))
    idx_ref[...] = (num_experts - 1) - (sb & (num_experts - 1))

    @pl.when(i == 0)
    def _():
        conf_ref[0, 0] = 0.0

    conf_ref[0, 0] += jnp.sum(m0) * (1.0 / n_tokens)


def kernel(x, W, b):
    n_tokens, input_dim = x.shape
    num_experts = W.shape[0]
    bt = min(1024, n_tokens)
    grid = (n_tokens // bt,)

    wt = W.T
    b2 = b.reshape(1, num_experts)

    gates, idx, conf = pl.pallas_call(
        lambda *refs: _router_kernel(*refs, n_tokens=n_tokens,
                                     num_experts=num_experts),
        grid=grid,
        in_specs=[
            pl.BlockSpec((bt, input_dim), lambda i: (i, 0)),
            pl.BlockSpec((input_dim, num_experts), lambda i: (0, 0)),
            pl.BlockSpec((1, num_experts), lambda i: (0, 0)),
        ],
        out_specs=[
            pl.BlockSpec((bt, num_experts), lambda i: (i, 0)),
            pl.BlockSpec((bt, K), lambda i: (i, 0)),
            pl.BlockSpec((1, 1), lambda i: (0, 0),
                         memory_space=pltpu.SMEM),
        ],
        out_shape=[
            jax.ShapeDtypeStruct((n_tokens, num_experts), jnp.float32),
            jax.ShapeDtypeStruct((n_tokens, K), jnp.int32),
            jax.ShapeDtypeStruct((1, 1), jnp.float32),
        ],
        compiler_params=pltpu.CompilerParams(
            dimension_semantics=("arbitrary",)),
    )(x, wt, b2)

    return gates, idx, conf.reshape(())


# speedup vs baseline: 1.4379x; 1.0592x over previous
import jax
import jax.numpy as jnp
from jax.experimental import pallas as pl
from jax.experimental.pallas import tpu as pltpu

K = 8


def _to_monotone(bits):
    return bits ^ (jax.lax.shift_right_arithmetic(bits, 31) & 0x7FFFFFFF)


def _router_kernel(x_ref, wt_ref, b_ref, gates_ref, idx_ref, conf_ref,
                   *, n_tokens: int, num_experts: int):
    i = pl.program_id(0)
    logits = jnp.dot(x_ref[...], wt_ref[...],
                     preferred_element_type=jnp.float32)
    logits = logits + b_ref[...]

    lane = jax.lax.broadcasted_iota(jnp.int32, logits.shape, 1)
    bits = jax.lax.bitcast_convert_type(logits, jnp.int32)
    s = _to_monotone(bits)
    key_s = (s & -64) | (num_experts - 1 - lane)
    keys = jax.lax.bitcast_convert_type(_to_monotone(key_s), jnp.float32)

    neg = jnp.float32(-jnp.inf)
    work = keys
    kbests = []
    for _ in range(K):
        kb = jnp.max(work, axis=1, keepdims=True)
        work = jnp.where(work == kb, neg, work)
        kbests.append(kb)

    kmax = kbests[0]
    kth = kbests[K - 1]

    e = jnp.exp(keys - kmax)
    esel = jnp.where(keys >= kth, e, 0.0)
    denom = jnp.sum(esel, axis=1, keepdims=True)
    gates_ref[...] = esel * (1.0 / denom)

    kcat = jnp.concatenate(kbests, axis=1)
    sb = _to_monotone(jax.lax.bitcast_convert_type(kcat, jnp.int32))
    idx_ref[...] = (num_experts - 1) - (sb & (num_experts - 1))

    @pl.when(i == 0)
    def _():
        conf_ref[0, 0] = 0.0

    conf_ref[0, 0] += jnp.sum(kmax) * (1.0 / n_tokens)


def kernel(x, W, b):
    n_tokens, input_dim = x.shape
    num_experts = W.shape[0]
    bt = min(1024, n_tokens)
    grid = (n_tokens // bt,)

    wt = W.T
    b2 = b.reshape(1, num_experts)

    gates, idx, conf = pl.pallas_call(
        lambda *refs: _router_kernel(*refs, n_tokens=n_tokens,
                                     num_experts=num_experts),
        grid=grid,
        in_specs=[
            pl.BlockSpec((bt, input_dim), lambda i: (i, 0)),
            pl.BlockSpec((input_dim, num_experts), lambda i: (0, 0)),
            pl.BlockSpec((1, num_experts), lambda i: (0, 0)),
        ],
        out_specs=[
            pl.BlockSpec((bt, num_experts), lambda i: (i, 0)),
            pl.BlockSpec((bt, K), lambda i: (i, 0)),
            pl.BlockSpec((1, 1), lambda i: (0, 0),
                         memory_space=pltpu.SMEM),
        ],
        out_shape=[
            jax.ShapeDtypeStruct((n_tokens, num_experts), jnp.float32),
            jax.ShapeDtypeStruct((n_tokens, K), jnp.int32),
            jax.ShapeDtypeStruct((1, 1), jnp.float32),
        ],
        compiler_params=pltpu.CompilerParams(
            dimension_semantics=("arbitrary",)),
    )(x, wt, b2)

    return gates, idx, conf.reshape(())
